# baseline (speedup 1.0000x reference)
import functools

import jax
import jax.numpy as jnp
from jax import lax
from jax.experimental import pallas as pl
from jax.experimental.pallas import tpu as pltpu
from jax.experimental.pallas import tpu_sc as plsc

_BS, _MC, _ML, _V, _P, _S = 16, 8, 20, 10000, 50, 300
_NTOK = _BS * _MC * _ML
_NROW = _BS * _MC
_NC, _NS = 2, 16
_NW = _NC * _NS
_TPW = _NTOK // _NW
_NCHUNK = _TPW // 16

_CHUNK = 64
_NCHUNKS = _NTOK // _CHUNK
_NBUF = 4
_NGROUP = _NCHUNKS // _NBUF


def _sc_gather_body(pos_tab, gt_pos, xpos_out, gtp_v, rowp_v, outp_v, semp):
    wid = lax.axis_index("s") * _NC + lax.axis_index("c")
    base = wid * _TPW
    pltpu.sync_copy(gt_pos.at[pl.ds(base, _TPW)], gtp_v)
    for i in range(_NCHUNK):
        sl = pl.ds(i * 16, 16)
        tok = lax.iota(jnp.int32, 16) + (base + i * 16)
        rowp_v[sl] = tok * _P + gtp_v[sl]
    pltpu.async_copy(pos_tab.at[rowp_v], outp_v, semp).wait()
    pltpu.sync_copy(outp_v, xpos_out.at[pl.ds(base, _TPW)])


@functools.cache
def _sc_gather_kernel():
  return functools.partial(
    pl.kernel,
    mesh=plsc.VectorSubcoreMesh(core_axis_name="c", subcore_axis_name="s",
                                num_cores=_NC, num_subcores=_NS),
    out_type=jax.ShapeDtypeStruct((_NTOK,), jnp.float32),
    scratch_types=[
        pltpu.VMEM((_TPW,), jnp.int32),
        pltpu.VMEM((_TPW,), jnp.int32),
        pltpu.VMEM((_TPW,), jnp.float32),
        pltpu.SemaphoreType.DMA,
    ],
  )(_sc_gather_body)


def _stream_body(cap_hbm, pos_ref, gtc_ref,
                 lsec_ref, xcap_ref, lsep_ref,
                 bufs, sems):
    def copy(c, b):
        return pltpu.make_async_copy(
            cap_hbm.at[pl.ds(c * _CHUNK, _CHUNK), :],
            bufs.at[b],
            sems.at[b],
        )

    for b in range(_NBUF):
        copy(b, b).start()

    def group(g, _):
        for b in range(_NBUF):
            c = g * _NBUF + b
            copy(c, b).wait()
            x = bufs[b]
            m = jnp.max(x, axis=1, keepdims=True)
            s = jnp.sum(jnp.exp(x - m), axis=1, keepdims=True)
            lsec_ref[pl.ds(c * _CHUNK, _CHUNK), :] = jnp.log(s) + m
            gtc = gtc_ref[pl.ds(c * _CHUNK, _CHUNK), :]
            v = lax.broadcasted_iota(jnp.int32, (_CHUNK, _V), 1)
            xcap_ref[pl.ds(c * _CHUNK, _CHUNK), :] = jnp.sum(
                jnp.where(v == gtc, x, 0.0), axis=1, keepdims=True)

            @pl.when(g < _NGROUP - 1)
            def _():
                copy(c + _NBUF, b).start()
        return ()

    lax.fori_loop(0, _NGROUP, group, (), unroll=False)

    xp = pos_ref[...]
    mp = jnp.max(xp, axis=1, keepdims=True)
    sp = jnp.sum(jnp.exp(xp - mp), axis=1, keepdims=True)
    lsep_ref[...] = jnp.log(sp) + mp


def _stream_call(cap2d, pos2d, gtc2d):
    oshape = jax.ShapeDtypeStruct((_NTOK, 1), jnp.float32)
    return pl.pallas_call(
        _stream_body,
        in_specs=[pl.BlockSpec(memory_space=pl.MemorySpace.ANY),
                  pl.BlockSpec(memory_space=pltpu.MemorySpace.VMEM),
                  pl.BlockSpec(memory_space=pltpu.MemorySpace.VMEM)],
        out_specs=[pl.BlockSpec(memory_space=pltpu.MemorySpace.VMEM)] * 3,
        out_shape=[oshape] * 3,
        scratch_shapes=[
            pltpu.VMEM((_NBUF, _CHUNK, _V), jnp.float32),
            pltpu.SemaphoreType.DMA((_NBUF,)),
        ],
    )(cap2d, pos2d, gtc2d)


def _combine_body(lsec_ref, xcap_ref, lsep_ref, xpos_ref, lens_ref, cnt_ref,
                  sem_x_ref, sem_y_ref, out_ref):
    lsec = lsec_ref[...]
    xcap = xcap_ref[...]
    lsep = lsep_ref[...]
    xpos = xpos_ref[...]
    lens = lens_ref[...]
    kk = lax.broadcasted_iota(jnp.int32, (_NROW, _BS), 1)
    bb = lax.broadcasted_iota(jnp.int32, (_NROW, _BS), 0) // _MC
    cnt_row = jnp.sum(jnp.where(kk == bb, cnt_ref[...], 0), axis=1,
                      keepdims=True)
    c_idx = lax.broadcasted_iota(jnp.int32, (_NROW, 1), 0) % _MC
    capmask = c_idx < cnt_row
    t = lax.broadcasted_iota(jnp.int32, (_NROW, _ML), 1)
    tokf = ((t < lens) & capmask).astype(jnp.float32)
    ntok = jnp.sum(tokf)
    cap_loss = jnp.sum((lsec - xcap) * tokf) / ntok
    pos_loss = jnp.sum((lsep - xpos) * tokf) / ntok
    x = sem_x_ref[...]
    y = sem_y_ref[...]
    bce = jnp.maximum(x, 0.0) - x * y + jnp.log1p(jnp.exp(-jnp.abs(x)))
    capf = capmask.astype(jnp.float32)
    sem_loss = jnp.sum(bce * capf) / (jnp.sum(capf) * _S)
    out_ref[0] = cap_loss + sem_loss + pos_loss
    out_ref[1] = cap_loss
    out_ref[2] = sem_loss
    out_ref[3] = pos_loss


def _combine_call(lsec, xcap, lsep, xpos, lens, cnt, sem_x, sem_y):
    return pl.pallas_call(
        _combine_body,
        out_specs=pl.BlockSpec(memory_space=pltpu.MemorySpace.SMEM),
        out_shape=jax.ShapeDtypeStruct((4,), jnp.float32),
    )(lsec, xcap, lsep, xpos, lens, cnt, sem_x, sem_y)


def kernel(gt_captions, gt_cap_lens, pred_captions, gt_caps_sem_enc,
           pred_caps_sem_enc, gt_pos_seq, pred_pos_seq, gt_program,
           gt_prog_len, pred_program, gt_intervals, pred_intervals,
           gt_proposals, pred_proposals, gt_caps_count, pred_caps_count,
           gt_proposals_count):
    cap2d = pred_captions.reshape(_NTOK, _V)
    pos2d = pred_pos_seq.reshape(_NTOK, _P)
    pos_tab = pred_pos_seq.reshape(_NTOK * _P)
    gtc2d = gt_captions.reshape(_NTOK, 1).astype(jnp.int32)
    gt_posf = gt_pos_seq.reshape(_NTOK).astype(jnp.int32)

    xpos = _sc_gather_kernel()(pos_tab, gt_posf)
    lsec, xcap, lsep = _stream_call(cap2d, pos2d, gtc2d)

    out = _combine_call(
        lsec.reshape(_NROW, _ML), xcap.reshape(_NROW, _ML),
        lsep.reshape(_NROW, _ML), xpos.reshape(_NROW, _ML),
        gt_cap_lens.reshape(_NROW, 1).astype(jnp.int32),
        gt_caps_count.reshape(1, _BS).astype(jnp.int32),
        pred_caps_sem_enc.reshape(_NROW, _S),
        gt_caps_sem_enc.reshape(_NROW, _S),
    )
    return (out[0], out[1], out[2], out[3])
